# baseline (speedup 1.0000x reference)
import functools

import jax
import jax.numpy as jnp
from jax import lax
from jax.experimental import pallas as pl
from jax.experimental.pallas import tpu as pltpu
from jax.experimental.pallas import tpu_sc as plsc

_N = 10000
_NP = 10112
_E = 320000
_D = 128
_NC = 2
_NS = 16
_NT = _NC * _NS
_ROWS_PER_TILE = _NP // _NS
_KP = 128
_NCHT = _E // _KP
_NCH0 = _NCHT // _NT
_NXT = _NCHT - _NCH0 * _NT
_NBUF = 3
_STEPS = _NCH0 + 1 + 3
_NGRP = (_STEPS + _NBUF - 1) // _NBUF


def _scale_body(x_ref, d_ref, o_ref):
    dcol = jnp.transpose(d_ref[...], (1, 0))
    o_ref[...] = x_ref[...] * dcol


def _combine_body(s_ref, p0_ref, p1_ref, d_ref, h0_ref, w_ref, o_ref):
    theta = s_ref[0]
    alpha = s_ref[1]
    agg = p0_ref[0] + p1_ref[0]
    dcol = jnp.transpose(d_ref[...], (1, 0))
    support = (1.0 - alpha) * (agg * dcol) + alpha * h0_ref[...]
    o_ref[...] = theta * jnp.dot(
        support, w_ref[...], preferred_element_type=jnp.float32
    ) + (1.0 - theta) * support


def _sc_segment_sum_body(h_hbm, ei_hbm, out_hbm,
                         is0, is1, is2,
                         r0, r1, r2, agg_sh,
                         q0, q1, q2, g0, g1, g2):
    c = lax.axis_index("c")
    s = lax.axis_index("s")
    wid = c * _NS + s
    ibs = (is0, is1, is2)
    rows = (r0, r1, r2)
    isem = (q0, q1, q2)
    gsem = (g0, g1, g2)

    def _zero_row(i, carry):
        for j in range(_D // 16):
            r0[i, pl.ds(j * 16, 16)] = jnp.zeros((16,), jnp.float32)
        return carry

    lax.fori_loop(0, _KP, _zero_row, 0)
    row0 = s * _ROWS_PER_TILE
    nfull = _ROWS_PER_TILE // _KP
    for i in range(nfull):
        pltpu.sync_copy(r0, agg_sh.at[pl.ds(row0 + i * _KP, _KP), :])
    rem = _ROWS_PER_TILE - nfull * _KP
    if rem:
        pltpu.sync_copy(r0.at[pl.ds(0, rem), :],
                        agg_sh.at[pl.ds(row0 + nfull * _KP, rem), :])
    plsc.subcore_barrier()

    nch = jnp.where(wid < _NXT, _NCH0 + 1, _NCH0)
    cb = wid * _NCH0 + jnp.minimum(wid, _NXT)

    def _group(g, carry):
        for b in range(_NBUF):
            step = g * _NBUF + b
            bp = (b + 2) % _NBUF

            @pl.when(jnp.logical_and(step >= 3, step < nch + 3))
            def _():
                pltpu.make_async_copy(
                    h_hbm.at[ibs[b].at[0]], rows[b], gsem[b]).wait()
                pltpu.sync_copy(rows[b], agg_sh.at[ibs[b].at[1]], add=True)

            @pl.when(step < nch)
            def _():
                off = pl.multiple_of((cb + step) * _KP, 128)
                pltpu.async_copy(ei_hbm.at[:, pl.ds(off, _KP)], ibs[b], isem[b])

            @pl.when(jnp.logical_and(step >= 1, step < nch + 1))
            def _():
                off = pl.multiple_of((cb + step - 1) * _KP, 128)
                pltpu.make_async_copy(
                    ei_hbm.at[:, pl.ds(off, _KP)], ibs[bp], isem[bp]).wait()
                pltpu.async_copy(h_hbm.at[ibs[bp].at[0]], rows[bp], gsem[bp])
        return carry

    lax.fori_loop(0, _NGRP, _group, 0)
    plsc.subcore_barrier()

    o0 = pl.multiple_of(c * _NP + row0, 8)
    pltpu.sync_copy(agg_sh.at[pl.ds(row0, _ROWS_PER_TILE), :],
                    out_hbm.at[pl.ds(o0, _ROWS_PER_TILE), :])


_sc_segment_sum = functools.partial(
    pl.kernel,
    out_type=jax.ShapeDtypeStruct((_NC * _NP, _D), jnp.float32),
    mesh=plsc.VectorSubcoreMesh(
        core_axis_name="c", subcore_axis_name="s",
        num_cores=_NC, num_subcores=_NS),
    scratch_types=[
        pltpu.VMEM((2, _KP), jnp.int32),
        pltpu.VMEM((2, _KP), jnp.int32),
        pltpu.VMEM((2, _KP), jnp.int32),
        pltpu.VMEM((_KP, _D), jnp.float32),
        pltpu.VMEM((_KP, _D), jnp.float32),
        pltpu.VMEM((_KP, _D), jnp.float32),
        pltpu.VMEM_SHARED((_NP, _D), jnp.float32),
        pltpu.SemaphoreType.DMA,
        pltpu.SemaphoreType.DMA,
        pltpu.SemaphoreType.DMA,
        pltpu.SemaphoreType.DMA,
        pltpu.SemaphoreType.DMA,
        pltpu.SemaphoreType.DMA,
    ],
)(_sc_segment_sum_body)


def kernel(in_feat, edge_index, d, h0, W, lamda, alpha, l):
    ei = edge_index.astype(jnp.int32)
    dr = d[None, :]
    theta = jnp.log(lamda / l + 1.0)
    scals = jnp.stack([theta, alpha]).astype(jnp.float32)

    h = pl.pallas_call(
        _scale_body,
        grid=(1,),
        in_specs=[
            pl.BlockSpec((_N, _D), lambda i: (0, 0)),
            pl.BlockSpec((1, _N), lambda i: (0, 0)),
        ],
        out_specs=pl.BlockSpec((_N, _D), lambda i: (0, 0)),
        out_shape=jax.ShapeDtypeStruct((_N, _D), jnp.float32),
    )(in_feat, dr)

    partials = _sc_segment_sum(h, ei).reshape(_NC, _NP, _D)

    out = pl.pallas_call(
        _combine_body,
        grid=(1,),
        in_specs=[
            pl.BlockSpec(memory_space=pltpu.SMEM),
            pl.BlockSpec((1, _N, _D), lambda i: (0, 0, 0)),
            pl.BlockSpec((1, _N, _D), lambda i: (1, 0, 0)),
            pl.BlockSpec((1, _N), lambda i: (0, 0)),
            pl.BlockSpec((_N, _D), lambda i: (0, 0)),
            pl.BlockSpec((_D, _D), lambda i: (0, 0)),
        ],
        out_specs=pl.BlockSpec((_N, _D), lambda i: (0, 0)),
        out_shape=jax.ShapeDtypeStruct((_N, _D), jnp.float32),
    )(scals, partials, partials, dr, h0, W)
    return out
